# baseline (speedup 1.0000x reference)
import functools

import jax
import jax.numpy as jnp
from jax import lax
from jax.experimental import pallas as pl
from jax.experimental.pallas import tpu as pltpu
from jax.experimental.pallas import tpu_sc as plsc

N_NODES = 10000
N_EDGES = 320000
D = 128
LANES = 16

NUM_CORES = 2
NUM_SUBCORES = 16
NUM_WORKERS = NUM_CORES * NUM_SUBCORES
EDGES_PER_WORKER = N_EDGES // NUM_WORKERS
BATCH = 80
NB = EDGES_PER_WORKER // BATCH
NBP = 128
CHUNK = 32
NSTAGE = NBP // CHUNK
N_PAD = 10240
ROWS_PER_TILE = N_PAD // NUM_SUBCORES
GROUPS = BATCH // LANES
BLOCKS = D // LANES


def _sc_segment_sum(features, esrc, edst, evals):
  mesh = plsc.VectorSubcoreMesh(core_axis_name="c", subcore_axis_name="s")

  @functools.partial(
      pl.kernel,
      mesh=mesh,
      out_type=jax.ShapeDtypeStruct((NUM_CORES, N_PAD, D), jnp.float32),
      scratch_types=[
          pltpu.VMEM((CHUNK * BATCH,), jnp.int32),
          pltpu.VMEM((CHUNK * BATCH,), jnp.int32),
          pltpu.VMEM((CHUNK * BATCH,), jnp.float32),
          pltpu.VMEM((BATCH,), jnp.int32),
          pltpu.VMEM((BATCH,), jnp.int32),
          pltpu.VMEM((BATCH, D), jnp.float32),
          pltpu.VMEM_SHARED((N_PAD, D), jnp.float32),
          pltpu.SemaphoreType.DMA,
      ],
  )
  def k(feat_hbm, esrc_hbm, edst_hbm, eval_hbm, out_hbm,
        src_big, dst_big, val_big, src_v, dst_v, rows_v, accum, sem):
    c = lax.axis_index("c")
    s = lax.axis_index("s")
    wid = s * NUM_CORES + c

    zeros = jnp.zeros((LANES,), jnp.float32)

    def zero_body(i, carry):
      for j in range(BLOCKS):
        rows_v[i, pl.ds(j * LANES, LANES)] = zeros
      return carry

    lax.fori_loop(0, BATCH, zero_body, None)
    for kk in range(ROWS_PER_TILE // BATCH):
      pltpu.sync_copy(
          rows_v.at[pl.ds(0, BATCH)],
          accum.at[pl.ds(s * ROWS_PER_TILE + kk * BATCH, BATCH)])
    plsc.subcore_barrier()

    cbytes = CHUNK * BATCH

    def batch_body(tb, carry2):
        jb = lax.rem(tb, CHUNK)
        bo = jb * BATCH

        @pl.when(jb == 0)
        def _stage():
          off = wid * (NBP * BATCH) + lax.div(tb, CHUNK) * cbytes
          pltpu.sync_copy(esrc_hbm.at[pl.ds(off, cbytes)], src_big)
          pltpu.sync_copy(edst_hbm.at[pl.ds(off, cbytes)], dst_big)
          pltpu.sync_copy(eval_hbm.at[pl.ds(off, cbytes)], val_big)
        for g in range(GROUPS):
          sl = pl.ds(g * LANES, LANES)
          src_v[sl] = src_big[pl.ds(bo + g * LANES, LANES)]
          dst_v[sl] = dst_big[pl.ds(bo + g * LANES, LANES)]
        pltpu.async_copy(feat_hbm.at[src_v], rows_v, sem).wait()

        def scale_group(g, carry3):
          vv = val_big[pl.ds(bo + g * LANES, LANES)]
          for lane in range(LANES):
            v = vv[lane]
            e = g * LANES + lane
            for j in range(BLOCKS):
              sl = pl.ds(j * LANES, LANES)
              rows_v[e, sl] = rows_v[e, sl] * v
          return carry3

        lax.fori_loop(0, GROUPS, scale_group, None)
        pltpu.sync_copy(rows_v, accum.at[dst_v], add=True)
        return carry2

    lax.fori_loop(0, NBP, batch_body, None)
    plsc.subcore_barrier()

    base = s * ROWS_PER_TILE
    pltpu.sync_copy(accum.at[pl.ds(base, ROWS_PER_TILE)],
                    out_hbm.at[c, pl.ds(base, ROWS_PER_TILE)])

  return k(features, esrc, edst, evals)


def _tc_combine_matmul(p0, p1, w):
  block_rows = 1000

  def body(p0_ref, p1_ref, w_ref, out_ref):
    out_ref[...] = jnp.dot(p0_ref[...] + p1_ref[...], w_ref[...],
                           preferred_element_type=jnp.float32)

  return pl.pallas_call(
      body,
      grid=(N_NODES // block_rows,),
      in_specs=[
          pl.BlockSpec((block_rows, D), lambda i: (i, 0)),
          pl.BlockSpec((block_rows, D), lambda i: (i, 0)),
          pl.BlockSpec((D, D), lambda i: (0, 0)),
      ],
      out_specs=pl.BlockSpec((block_rows, D), lambda i: (i, 0)),
      out_shape=jax.ShapeDtypeStruct((N_NODES, D), jnp.float32),
  )(p0, p1, w)


def _pack_edata(edge_index, edge_values):
  pad_rows = NBP - NB
  src = edge_index[0].reshape(NUM_WORKERS, NB, BATCH)
  dst = edge_index[1].reshape(NUM_WORKERS, NB, BATCH)
  vals = edge_values.reshape(NUM_WORKERS, NB, BATCH)
  zpad = jnp.zeros((NUM_WORKERS, pad_rows, BATCH), jnp.int32)
  esrc = jnp.concatenate([src, zpad], axis=1).reshape(-1)
  edst = jnp.concatenate([dst, zpad], axis=1).reshape(-1)
  evals = jnp.concatenate([vals, zpad.astype(jnp.float32)],
                          axis=1).reshape(-1)
  return esrc, edst, evals


def kernel(features, edge_index, edge_values, W):
  esrc, edst, evals = _pack_edata(edge_index, edge_values)
  partials = _sc_segment_sum(features, esrc, edst, evals)
  return _tc_combine_matmul(partials[0], partials[1], W)


# speedup vs baseline: 1.7076x; 1.7076x over previous
import functools

import jax
import jax.numpy as jnp
from jax import lax
from jax.experimental import pallas as pl
from jax.experimental.pallas import tpu as pltpu
from jax.experimental.pallas import tpu_sc as plsc

N_NODES = 10000
N_EDGES = 320000
D = 128
LANES = 16

NUM_CORES = 2
NUM_SUBCORES = 16
NUM_WORKERS = NUM_CORES * NUM_SUBCORES
EDGES_PER_WORKER = N_EDGES // NUM_WORKERS
BATCH = 80
NB = EDGES_PER_WORKER // BATCH
N_PAD = 10240
ROWS_PER_TILE = N_PAD // NUM_SUBCORES
GROUPS = BATCH // LANES
BLOCKS = D // LANES


def _sc_segment_sum(features, esrc, edst, evals):
  mesh = plsc.VectorSubcoreMesh(core_axis_name="c", subcore_axis_name="s")

  @functools.partial(
      pl.kernel,
      mesh=mesh,
      out_type=jax.ShapeDtypeStruct((NUM_CORES, N_PAD, D), jnp.float32),
      scratch_types=[
          pltpu.VMEM((BATCH,), jnp.int32),
          pltpu.VMEM((BATCH,), jnp.int32),
          pltpu.VMEM((BATCH,), jnp.float32),
          pltpu.VMEM((BATCH, D), jnp.float32),
          pltpu.VMEM_SHARED((N_PAD, D), jnp.float32),
          pltpu.SemaphoreType.DMA,
          pltpu.SemaphoreType.DMA,
      ],
  )
  def k(feat_hbm, esrc_hbm, edst_hbm, eval_hbm, out_hbm,
        src_v, dst_v, val_v, rows_v, accum, lsem, gsem):
    c = lax.axis_index("c")
    s = lax.axis_index("s")
    wid = s * NUM_CORES + c
    ebase = wid * EDGES_PER_WORKER

    zeros = jnp.zeros((LANES,), jnp.float32)

    def zero_body(i, carry):
      for j in range(BLOCKS):
        rows_v[i, pl.ds(j * LANES, LANES)] = zeros
      return carry

    lax.fori_loop(0, BATCH, zero_body, None)
    for kk in range(ROWS_PER_TILE // BATCH):
      pltpu.sync_copy(
          rows_v.at[pl.ds(0, BATCH)],
          accum.at[pl.ds(s * ROWS_PER_TILE + kk * BATCH, BATCH)])
    plsc.subcore_barrier()

    def batch_body(t, carry):
      off = ebase + t * BATCH
      cs = pltpu.async_copy(esrc_hbm.at[pl.ds(off, BATCH)], src_v, lsem)
      cd = pltpu.async_copy(edst_hbm.at[pl.ds(off, BATCH)], dst_v, lsem)
      cv = pltpu.async_copy(eval_hbm.at[pl.ds(off, BATCH)], val_v, lsem)
      cs.wait()
      cd.wait()
      cv.wait()
      pltpu.async_copy(feat_hbm.at[src_v], rows_v, gsem).wait()

      def scale_group(g, carry2):
        vv = val_v[pl.ds(g * LANES, LANES)]
        for lane in range(LANES):
          v = vv[lane]
          e = g * LANES + lane
          for j in range(BLOCKS):
            sl = pl.ds(j * LANES, LANES)
            rows_v[e, sl] = rows_v[e, sl] * v
        return carry2

      lax.fori_loop(0, GROUPS, scale_group, None)
      pltpu.sync_copy(rows_v, accum.at[dst_v], add=True)
      return carry

    lax.fori_loop(0, NB, batch_body, None)
    plsc.subcore_barrier()

    base = s * ROWS_PER_TILE
    pltpu.sync_copy(accum.at[pl.ds(base, ROWS_PER_TILE)],
                    out_hbm.at[c, pl.ds(base, ROWS_PER_TILE)])

  return k(features, esrc, edst, evals)


def _tc_combine_matmul(p0, p1, w):
  block_rows = 1000

  def body(p0_ref, p1_ref, w_ref, out_ref):
    out_ref[...] = jnp.dot(p0_ref[...] + p1_ref[...], w_ref[...],
                           preferred_element_type=jnp.float32)

  return pl.pallas_call(
      body,
      grid=(N_NODES // block_rows,),
      in_specs=[
          pl.BlockSpec((block_rows, D), lambda i: (i, 0)),
          pl.BlockSpec((block_rows, D), lambda i: (i, 0)),
          pl.BlockSpec((D, D), lambda i: (0, 0)),
      ],
      out_specs=pl.BlockSpec((block_rows, D), lambda i: (i, 0)),
      out_shape=jax.ShapeDtypeStruct((N_NODES, D), jnp.float32),
  )(p0, p1, w)


def kernel(features, edge_index, edge_values, W):
  partials = _sc_segment_sum(features, edge_index[0], edge_index[1],
                             edge_values)
  return _tc_combine_matmul(partials[0], partials[1], W)


# speedup vs baseline: 1.7222x; 1.0086x over previous
import functools

import jax
import jax.numpy as jnp
from jax import lax
from jax.experimental import pallas as pl
from jax.experimental.pallas import tpu as pltpu
from jax.experimental.pallas import tpu_sc as plsc

N_NODES = 10000
N_EDGES = 320000
D = 128
LANES = 16

NUM_CORES = 2
NUM_SUBCORES = 16
NUM_WORKERS = NUM_CORES * NUM_SUBCORES
EDGES_PER_WORKER = N_EDGES // NUM_WORKERS
BATCH = 80
NB = EDGES_PER_WORKER // BATCH
N_PAD = 10240
ROWS_PER_TILE = N_PAD // NUM_SUBCORES
GROUPS = BATCH // LANES
BLOCKS = D // LANES


def _sc_segment_sum(features, esrc, edst, evals):
  mesh = plsc.VectorSubcoreMesh(core_axis_name="c", subcore_axis_name="s")

  @functools.partial(
      pl.kernel,
      mesh=mesh,
      out_type=jax.ShapeDtypeStruct((NUM_CORES, N_PAD, D), jnp.float32),
      scratch_types=[
          pltpu.VMEM((BATCH,), jnp.int32),
          pltpu.VMEM((BATCH,), jnp.int32),
          pltpu.VMEM((BATCH,), jnp.float32),
          pltpu.VMEM((BATCH, D), jnp.float32),
          pltpu.VMEM_SHARED((N_PAD, D), jnp.float32),
          pltpu.SemaphoreType.DMA,
          pltpu.SemaphoreType.DMA,
      ],
  )
  def k(feat_hbm, esrc_hbm, edst_hbm, eval_hbm, out_hbm,
        src_v, dst_v, val_v, rows_v, accum, lsem, gsem):
    c = lax.axis_index("c")
    s = lax.axis_index("s")
    wid = s * NUM_CORES + c
    ebase = wid * EDGES_PER_WORKER

    zeros = jnp.zeros((LANES,), jnp.float32)

    def zero_body(i, carry):
      for j in range(BLOCKS):
        rows_v[i, pl.ds(j * LANES, LANES)] = zeros
      return carry

    lax.fori_loop(0, BATCH, zero_body, None)
    for kk in range(ROWS_PER_TILE // BATCH):
      pltpu.sync_copy(
          rows_v.at[pl.ds(0, BATCH)],
          accum.at[pl.ds(s * ROWS_PER_TILE + kk * BATCH, BATCH)])
    plsc.subcore_barrier()

    def batch_body(t, carry):
      off = ebase + t * BATCH
      cs = pltpu.async_copy(esrc_hbm.at[pl.ds(off, BATCH)], src_v, lsem)
      cd = pltpu.async_copy(edst_hbm.at[pl.ds(off, BATCH)], dst_v, lsem)
      cv = pltpu.async_copy(eval_hbm.at[pl.ds(off, BATCH)], val_v, lsem)
      cs.wait()
      cg = pltpu.async_copy(feat_hbm.at[src_v], rows_v, gsem)
      cd.wait()
      cv.wait()
      cg.wait()

      def scale_group(g, carry2):
        vv = val_v[pl.ds(g * LANES, LANES)]
        for lane in range(LANES):
          v = vv[lane]
          e = g * LANES + lane
          for j in range(BLOCKS):
            sl = pl.ds(j * LANES, LANES)
            rows_v[e, sl] = rows_v[e, sl] * v
        return carry2

      lax.fori_loop(0, GROUPS, scale_group, None)
      pltpu.sync_copy(rows_v, accum.at[dst_v], add=True)
      return carry

    lax.fori_loop(0, NB, batch_body, None)
    plsc.subcore_barrier()

    base = s * ROWS_PER_TILE
    pltpu.sync_copy(accum.at[pl.ds(base, ROWS_PER_TILE)],
                    out_hbm.at[c, pl.ds(base, ROWS_PER_TILE)])

  return k(features, esrc, edst, evals)


def _tc_combine_matmul(p0, p1, w):
  block_rows = 1000

  def body(p0_ref, p1_ref, w_ref, out_ref):
    out_ref[...] = jnp.dot(p0_ref[...] + p1_ref[...], w_ref[...],
                           preferred_element_type=jnp.float32)

  return pl.pallas_call(
      body,
      grid=(N_NODES // block_rows,),
      in_specs=[
          pl.BlockSpec((block_rows, D), lambda i: (i, 0)),
          pl.BlockSpec((block_rows, D), lambda i: (i, 0)),
          pl.BlockSpec((D, D), lambda i: (0, 0)),
      ],
      out_specs=pl.BlockSpec((block_rows, D), lambda i: (i, 0)),
      out_shape=jax.ShapeDtypeStruct((N_NODES, D), jnp.float32),
  )(p0, p1, w)


def kernel(features, edge_index, edge_values, W):
  partials = _sc_segment_sum(features, edge_index[0], edge_index[1],
                             edge_values)
  return _tc_combine_matmul(partials[0], partials[1], W)
